# Initial kernel scaffold; baseline (speedup 1.0000x reference)
import functools

import jax
import jax.numpy as jnp
from jax import lax
from jax.experimental import pallas as pl
from jax.experimental.pallas import tpu as pltpu
from jax.experimental.pallas import tpu_sc as plsc

N_NODES = 10000
N_EDGES = 320000
D_IN = 128
D_HID = 128
D_OUT = 64
N_GRAPHS = 64

NC = 2
NS = 16
L = 16
NW = NC * NS
TILE_E = N_EDGES // NW
BLK = 80
NBLK = TILE_E // BLK
ROWS_PER_TILE = N_NODES // NS
NCHUNK = D_HID // L


def _proj_body(x_ref, wl_ref, wr_ref, xl_ref, xr_ref):
    x = x_ref[...]
    xl_ref[...] = jnp.dot(x, wl_ref[...], preferred_element_type=jnp.float32)
    xr_ref[...] = jnp.dot(x, wr_ref[...], preferred_element_type=jnp.float32)


_proj = pl.pallas_call(
    _proj_body,
    out_shape=(
        jax.ShapeDtypeStruct((N_NODES, D_HID), jnp.float32),
        jax.ShapeDtypeStruct((N_NODES, D_HID), jnp.float32),
    ),
)


def _edge_body(xl_hbm, xr_hbm, src_hbm, dst_hbm, att_hbm, zrow_hbm, z16_hbm,
               num_out, den_out,
               src_v, dst_v, xlr, xrr, accb, denb, wb, attv,
               sh_num, sh_den, sem1, sem2):
    cid = lax.axis_index("c")
    sid = lax.axis_index("s")
    wid = sid * NC + cid

    row0 = sid * ROWS_PER_TILE
    pltpu.sync_copy(zrow_hbm, sh_num.at[pl.ds(row0, ROWS_PER_TILE)])
    pltpu.sync_copy(z16_hbm, sh_den.at[pl.ds(row0, ROWS_PER_TILE)])
    pltpu.sync_copy(z16_hbm.at[pl.ds(0, BLK)], denb)
    pltpu.sync_copy(att_hbm, attv)
    plsc.subcore_barrier()

    att_chunks = [attv[pl.ds(c * L, L)] for c in range(NCHUNK)]
    iota = lax.iota(jnp.int32, L)
    zeros_i = jnp.zeros((L,), jnp.int32)
    ebase = wid * TILE_E

    def block_body(j, carry):
        e0 = ebase + j * BLK
        pltpu.sync_copy(src_hbm.at[pl.ds(e0, BLK)], src_v)
        pltpu.sync_copy(dst_hbm.at[pl.ds(e0, BLK)], dst_v)
        d1 = pltpu.async_copy(xl_hbm.at[src_v], xlr, sem1)
        d2 = pltpu.async_copy(xr_hbm.at[dst_v], xrr, sem2)
        d1.wait()
        d2.wait()

        def score_body(i, c0):
            acc = jnp.zeros((L,), jnp.float32)
            for c in range(NCHUNK):
                s = xlr[i, pl.ds(c * L, L)] + xrr[i, pl.ds(c * L, L)]
                acc = acc + att_chunks[c] * jnp.maximum(s, 0.2 * s)
            accb[i, :] = acc
            return c0

        lax.fori_loop(0, BLK, score_body, 0)

        for g in range(BLK // L):
            rows = g * L + iota
            e16 = jnp.zeros((L,), jnp.float32)
            for c in range(L):
                e16 = e16 + plsc.load_gather(accb, [rows, zeros_i + c])
            w16 = jnp.exp(e16)
            wb[pl.ds(g * L, L)] = w16
            plsc.store_scatter(denb, [rows, zeros_i], w16)

        def wmul_body(i, c0):
            w = plsc.load_gather(wb, [zeros_i + i])
            for c in range(NCHUNK):
                xlr[i, pl.ds(c * L, L)] = xlr[i, pl.ds(c * L, L)] * w
            return c0

        lax.fori_loop(0, BLK, wmul_body, 0)

        pltpu.sync_copy(xlr, sh_num.at[dst_v], add=True)
        pltpu.sync_copy(denb, sh_den.at[dst_v], add=True)
        return carry

    lax.fori_loop(0, NBLK, block_body, 0)
    plsc.subcore_barrier()

    pltpu.sync_copy(sh_num.at[pl.ds(row0, ROWS_PER_TILE)],
                    num_out.at[cid, pl.ds(row0, ROWS_PER_TILE)])
    pltpu.sync_copy(sh_den.at[pl.ds(row0, ROWS_PER_TILE)],
                    den_out.at[cid, pl.ds(row0, ROWS_PER_TILE)])


_edge_kernel = functools.partial(
    pl.kernel,
    out_type=(
        jax.ShapeDtypeStruct((NC, N_NODES, D_HID), jnp.float32),
        jax.ShapeDtypeStruct((NC, N_NODES, 16), jnp.float32),
    ),
    mesh=plsc.VectorSubcoreMesh(core_axis_name="c", subcore_axis_name="s"),
    scratch_types=[
        pltpu.VMEM((BLK,), jnp.int32),
        pltpu.VMEM((BLK,), jnp.int32),
        pltpu.VMEM((BLK, D_HID), jnp.float32),
        pltpu.VMEM((BLK, D_HID), jnp.float32),
        pltpu.VMEM((BLK, 16), jnp.float32),
        pltpu.VMEM((BLK, 16), jnp.float32),
        pltpu.VMEM((BLK,), jnp.float32),
        pltpu.VMEM((D_HID,), jnp.float32),
        pltpu.VMEM_SHARED((N_NODES, D_HID), jnp.float32),
        pltpu.VMEM_SHARED((N_NODES, 16), jnp.float32),
        pltpu.SemaphoreType.DMA,
        pltpu.SemaphoreType.DMA,
    ],
)(_edge_body)


def _final_body(num_ref, den_ref, batch_ref, bconv_ref, fcw_ref, fcb_ref,
                out_ref):
    num = num_ref[0] + num_ref[1]
    den = den_ref[0, :, 0:1] + den_ref[1, :, 0:1]
    out = num / (den + 1e-16) + bconv_ref[...]
    gids = lax.broadcasted_iota(jnp.int32, (N_GRAPHS, N_NODES), 0)
    m = (batch_ref[...] == gids).astype(jnp.float32)
    sums = jnp.dot(m, out, preferred_element_type=jnp.float32)
    counts = jnp.sum(m, axis=1)[:, None]
    pooled = sums / jnp.maximum(counts, 1.0)
    h = jnp.where(pooled > 0, pooled, 0.01 * pooled)
    out_ref[...] = (jnp.dot(h, fcw_ref[...], preferred_element_type=jnp.float32)
                    + fcb_ref[...])


_final = pl.pallas_call(
    _final_body,
    out_shape=jax.ShapeDtypeStruct((N_GRAPHS, D_OUT), jnp.float32),
)


def kernel(x, edge_index, batch, add_features, W_l, W_r, att, b_conv, fc_W,
           fc_b):
    xl, xr = _proj(x, W_l, W_r)
    src = edge_index[0].astype(jnp.int32)
    dst = edge_index[1].astype(jnp.int32)
    zrow = jnp.zeros((ROWS_PER_TILE, D_HID), jnp.float32)
    z16 = jnp.zeros((ROWS_PER_TILE, 16), jnp.float32)
    num2, den2 = _edge_kernel(xl, xr, src, dst, att, zrow, z16)
    batch_row = batch.astype(jnp.int32).reshape(1, N_NODES)
    return _final(num2, den2, batch_row, b_conv.reshape(1, D_HID), fc_W,
                  fc_b.reshape(1, D_OUT))


# speedup vs baseline: 4.6544x; 4.6544x over previous
import functools

import jax
import jax.numpy as jnp
from jax import lax
from jax.experimental import pallas as pl
from jax.experimental.pallas import tpu as pltpu
from jax.experimental.pallas import tpu_sc as plsc

N_NODES = 10000
N_EDGES = 320000
D_IN = 128
D_HID = 128
D_OUT = 64
N_GRAPHS = 64

NC = 2
NS = 16
L = 16
NW = NC * NS
N_PAD = 10240
NODES_PER_W = N_PAD // NW
SEGE = 4000
NSEG = N_EDGES // SEGE
NGRP = SEGE // L
CAP = SEGE + 2 * L
NCHUNK = D_HID // L


def _proj_body(x_ref, wl_ref, wr_ref, xl_ref, xr_ref):
    x = x_ref[...]
    xl_ref[...] = jnp.dot(x, wl_ref[...], preferred_element_type=jnp.float32)
    xr_ref[...] = jnp.dot(x, wr_ref[...], preferred_element_type=jnp.float32)


_proj = pl.pallas_call(
    _proj_body,
    out_shape=(
        jax.ShapeDtypeStruct((N_NODES, D_HID), jnp.float32),
        jax.ShapeDtypeStruct((N_NODES, D_HID), jnp.float32),
    ),
)


def _edge_body(xl_hbm, xr_hbm, src_hbm, dst_hbm, att_hbm,
               num_out, den_out,
               segs, segd, csrc, cdst, sidx, didx, xlr, xrr,
               accb, wb, attv, num_l, den_l, sem1, sem2):
    cid = lax.axis_index("c")
    sid = lax.axis_index("s")
    wid = sid * NC + cid
    lo = wid * NODES_PER_W
    zf = jnp.zeros((L,), jnp.float32)
    zi = jnp.zeros((L,), jnp.int32)
    iota = lax.iota(jnp.int32, L)

    def znum_body(i, c0):
        num_l[pl.ds(i * L, L)] = zf
        return c0

    lax.fori_loop(0, NODES_PER_W * NCHUNK, znum_body, 0)

    def zden_body(i, c0):
        den_l[pl.ds(i * L, L)] = zf
        return c0

    lax.fori_loop(0, NODES_PER_W // L, zden_body, 0)

    def zidx_body(i, c0):
        csrc[pl.ds(i * L, L)] = zi
        cdst[pl.ds(i * L, L)] = zi
        return c0

    lax.fori_loop(0, CAP // L, zidx_body, 0)
    pltpu.sync_copy(att_hbm, attv)

    att_chunks = [attv[pl.ds(c * L, L)] for c in range(NCHUNK)]

    def seg_body(s, carry):
        e0 = s * SEGE
        pltpu.sync_copy(src_hbm.at[pl.ds(e0, SEGE)], segs)
        pltpu.sync_copy(dst_hbm.at[pl.ds(e0, SEGE)], segd)

        def scan_body(g, cnt):
            d16 = segd[pl.ds(g * L, L)]
            s16 = segs[pl.ds(g * L, L)]
            dl = d16 - lo
            mask = (dl >= 0) & (dl < NODES_PER_W)
            ones = jnp.where(mask, 1, 0)
            csum = plsc.cumsum(ones)
            pos = (zi + cnt) + csum - ones
            plsc.store_scatter(csrc, [pos], s16, mask=mask)
            plsc.store_scatter(cdst, [pos], dl, mask=mask)
            return cnt + lax.reduce_max(csum, (0,))

        cnt = lax.fori_loop(0, NGRP, scan_body, 0)

        def blk_body(b, c0):
            sidx[...] = csrc[pl.ds(b * L, L)]
            didx[...] = cdst[pl.ds(b * L, L)]
            d1 = pltpu.async_copy(xl_hbm.at[sidx], xlr, sem1)
            d2 = pltpu.async_copy(xr_hbm.at[didx], xrr, sem2)
            d1.wait()
            d2.wait()

            def score_body(i, c1):
                acc = zf
                for c in range(NCHUNK):
                    sv = xlr[i, pl.ds(c * L, L)] + xrr[i, pl.ds(c * L, L)]
                    acc = acc + att_chunks[c] * jnp.maximum(sv, 0.2 * sv)
                accb[i, :] = acc
                return c1

            lax.fori_loop(0, L, score_body, 0)

            e16 = zf
            for c in range(L):
                e16 = e16 + plsc.load_gather(accb, [iota, zi + c])
            u = e16 * (1.0 / 64.0)
            p = 1.0 + u * (1.0 + u * (0.5 + u * (
                (1.0 / 6.0) + u * ((1.0 / 24.0) + u * (1.0 / 120.0)))))
            for _sq in range(6):
                p = p * p
            valid = (b * L + iota) < cnt
            w16 = jnp.where(valid, p, 0.0)
            wb[...] = w16
            dl16 = didx[...]

            for lane in range(L):
                plsc.addupdate_scatter(den_l, [dl16], w16, mask=iota == lane)
                wbc = plsc.load_gather(wb, [zi + lane])
                dbase = plsc.load_gather(didx, [zi + lane]) * D_HID
                for c in range(NCHUNK):
                    chunk = wbc * xlr[lane, pl.ds(c * L, L)]
                    plsc.addupdate_scatter(num_l, [dbase + c * L + iota],
                                           chunk)
            return c0

        nblk = (cnt + (L - 1)) // L
        lax.fori_loop(0, nblk, blk_body, 0)
        return carry

    lax.fori_loop(0, NSEG, seg_body, 0)

    pltpu.sync_copy(num_l, num_out.at[pl.ds(lo * D_HID, NODES_PER_W * D_HID)])
    pltpu.sync_copy(den_l, den_out.at[pl.ds(lo, NODES_PER_W)])


_edge_kernel = functools.partial(
    pl.kernel,
    out_type=(
        jax.ShapeDtypeStruct((N_PAD * D_HID,), jnp.float32),
        jax.ShapeDtypeStruct((N_PAD,), jnp.float32),
    ),
    mesh=plsc.VectorSubcoreMesh(core_axis_name="c", subcore_axis_name="s"),
    compiler_params=pltpu.CompilerParams(needs_layout_passes=False),
    scratch_types=[
        pltpu.VMEM((SEGE,), jnp.int32),
        pltpu.VMEM((SEGE,), jnp.int32),
        pltpu.VMEM((CAP,), jnp.int32),
        pltpu.VMEM((CAP,), jnp.int32),
        pltpu.VMEM((L,), jnp.int32),
        pltpu.VMEM((L,), jnp.int32),
        pltpu.VMEM((L, D_HID), jnp.float32),
        pltpu.VMEM((L, D_HID), jnp.float32),
        pltpu.VMEM((L, L), jnp.float32),
        pltpu.VMEM((L,), jnp.float32),
        pltpu.VMEM((D_HID,), jnp.float32),
        pltpu.VMEM((NODES_PER_W * D_HID,), jnp.float32),
        pltpu.VMEM((NODES_PER_W,), jnp.float32),
        pltpu.SemaphoreType.DMA,
        pltpu.SemaphoreType.DMA,
    ],
)(_edge_body)


def _final_body(num_ref, den_ref, batch_ref, bconv_ref, fcw_ref, fcb_ref,
                out_ref):
    num = num_ref[:N_NODES]
    den = den_ref[:N_NODES]
    out = num / (den + 1e-16) + bconv_ref[...]
    gids = lax.broadcasted_iota(jnp.int32, (N_GRAPHS, N_NODES), 0)
    m = (batch_ref[...] == gids).astype(jnp.float32)
    sums = jnp.dot(m, out, preferred_element_type=jnp.float32)
    counts = jnp.dot(m, jnp.ones((N_NODES, 1), jnp.float32),
                     preferred_element_type=jnp.float32)
    pooled = sums / jnp.maximum(counts, 1.0)
    h = jnp.where(pooled > 0, pooled, 0.01 * pooled)
    out_ref[...] = (jnp.dot(h, fcw_ref[...], preferred_element_type=jnp.float32)
                    + fcb_ref[...])


_final = pl.pallas_call(
    _final_body,
    out_shape=jax.ShapeDtypeStruct((N_GRAPHS, D_OUT), jnp.float32),
)


def kernel(x, edge_index, batch, add_features, W_l, W_r, att, b_conv, fc_W,
           fc_b):
    xl, xr = _proj(x, W_l, W_r)
    src = edge_index[0].astype(jnp.int32)
    dst = edge_index[1].astype(jnp.int32)
    numf, denf = _edge_kernel(xl, xr, src, dst, att)
    num2 = numf.reshape(N_PAD, D_HID)
    den2 = denf.reshape(N_PAD, 1)
    batch_row = batch.astype(jnp.int32).reshape(1, N_NODES)
    return _final(num2, den2, batch_row, b_conv.reshape(1, D_HID), fc_W,
                  fc_b.reshape(1, D_OUT))
